# Initial kernel scaffold; baseline (speedup 1.0000x reference)
import jax
import jax.numpy as jnp
from jax.experimental import pallas as pl


def kernel(x, edge_index, edge_attr, batch, W1, EW1, b1, W2, EW2, b2, Wl, bl):
    raise NotImplementedError("write your pallas kernel here")


# speedup vs baseline: 6.1797x; 6.1797x over previous
import functools
import jax
import jax.numpy as jnp
from jax import lax
from jax.experimental import pallas as pl
from jax.experimental.pallas import tpu as pltpu
from jax.experimental.pallas import tpu_sc as plsc

N_NODES = 10000
N_PAD = 10240
E_EDGES = 320000
E_PAD = 327680
N_TILES = 32
N_CHUNKS = 80
CHUNK = 128
ROWS_PER_TILE = N_PAD // 16
BLK = 1024
GRID = N_PAD // BLK

@functools.cache
def _sc_mesh():
    return plsc.VectorSubcoreMesh(core_axis_name="c", subcore_axis_name="s")


def _zero_vmem(buf, ncols):
    nj = ncols // 16
    def zrow(r, carry):
        for j in range(nj):
            buf[r, pl.ds(16 * j, 16)] = jnp.zeros((16,), jnp.float32)
        return carry
    lax.fori_loop(0, 128, zrow, 0)


def _edge_scatter_body(table_hbm, ridx_hbm, cidx_hbm, out_hbm,
                       acc_sh, rows_v, ridx_v, cidx_v, sem):
    ci = lax.axis_index("c")
    si = lax.axis_index("s")
    wid = ci * 16 + si

    _zero_vmem(rows_v, 128)
    def zchunk(k, carry):
        r = si * ROWS_PER_TILE + k * 128
        pltpu.sync_copy(rows_v, acc_sh.at[pl.ds(r, 128)])
        return carry
    lax.fori_loop(0, ROWS_PER_TILE // 128, zchunk, 0)
    plsc.subcore_barrier()

    pltpu.sync_copy(ridx_hbm.at[wid], ridx_v)
    pltpu.sync_copy(cidx_hbm.at[wid], cidx_v)

    def echunk(c, carry):
        pltpu.async_copy(table_hbm.at[ridx_v.at[c]], rows_v, sem).wait()
        pltpu.sync_copy(rows_v, acc_sh.at[cidx_v.at[c]], add=True)
        return carry
    lax.fori_loop(0, N_CHUNKS, echunk, 0)
    plsc.subcore_barrier()

    def xchunk(k, carry):
        r = si * ROWS_PER_TILE + k * 128
        pltpu.sync_copy(acc_sh.at[pl.ds(r, 128)], rows_v)
        pltpu.sync_copy(rows_v, out_hbm.at[ci, pl.ds(r, 128)])
        return carry
    lax.fori_loop(0, ROWS_PER_TILE // 128, xchunk, 0)


@functools.cache
def _edge_scatter():
    return pl.kernel(
        _edge_scatter_body,
        out_type=jax.ShapeDtypeStruct((2, N_PAD, 128), jnp.float32),
        mesh=_sc_mesh(),
        scratch_types=[
            pltpu.VMEM_SHARED((N_PAD, 128), jnp.float32),
            pltpu.VMEM((CHUNK, 128), jnp.float32),
            pltpu.VMEM((N_CHUNKS, CHUNK), jnp.int32),
            pltpu.VMEM((N_CHUNKS, CHUNK), jnp.int32),
            pltpu.SemaphoreType.DMA,
        ],
    )


def _attr_scatter_body(eb_hbm, ridx_hbm, out_hbm,
                       acc_sh, ebuf, ridx_v):
    ci = lax.axis_index("c")
    si = lax.axis_index("s")
    wid = ci * 16 + si

    _zero_vmem(ebuf, 128)
    def zchunk(k, carry):
        pltpu.sync_copy(ebuf, acc_sh.at[pl.ds(si * ROWS_PER_TILE + k * 128, 128)])
        return carry
    lax.fori_loop(0, ROWS_PER_TILE // 128, zchunk, 0)
    plsc.subcore_barrier()

    pltpu.sync_copy(ridx_hbm.at[wid], ridx_v)

    def echunk(c, carry):
        pltpu.sync_copy(eb_hbm.at[wid, c], ebuf)
        pltpu.sync_copy(ebuf, acc_sh.at[ridx_v.at[c]], add=True)
        return carry
    lax.fori_loop(0, N_CHUNKS, echunk, 0)
    plsc.subcore_barrier()

    def xchunk(k, carry):
        r = si * ROWS_PER_TILE + k * 128
        pltpu.sync_copy(acc_sh.at[pl.ds(r, 128)], ebuf)
        pltpu.sync_copy(ebuf, out_hbm.at[ci, pl.ds(r, 128)])
        return carry
    lax.fori_loop(0, ROWS_PER_TILE // 128, xchunk, 0)


@functools.cache
def _attr_scatter():
    return pl.kernel(
        _attr_scatter_body,
        out_type=jax.ShapeDtypeStruct((2, N_PAD, 128), jnp.float32),
        mesh=_sc_mesh(),
        scratch_types=[
            pltpu.VMEM_SHARED((N_PAD, 128), jnp.float32),
            pltpu.VMEM((CHUNK, 128), jnp.float32),
            pltpu.VMEM((N_CHUNKS, CHUNK), jnp.int32),
        ],
    )


def _dense1_body(x_ref, s5a_ref, s5b_ref, w1_ref, b1_ref, ew1_ref, ew2_ref,
                 ynorm_ref, z1_ref, ea2_ref, normc_ref):
    x = x_ref[...]
    s5 = s5a_ref[...] + s5b_ref[...]
    deg = s5[:, 4:5]
    norm = jnp.where(deg > 0, lax.rsqrt(jnp.maximum(deg, 1e-30)), 0.0)
    w = w1_ref[...]
    y = jnp.dot(x, w[2], preferred_element_type=jnp.float32)
    ynorm_ref[...] = jnp.concatenate(
        [norm * y, jnp.zeros((BLK, 64), jnp.float32)], axis=1)
    a = w[0] + w[1] - w[2]
    ew1 = ew1_ref[0] + ew1_ref[1] + ew1_ref[2]
    ew2 = ew2_ref[0] + ew2_ref[1] + ew2_ref[2]
    s4 = s5[:, 0:4]
    z1_ref[...] = (jnp.dot(x, a, preferred_element_type=jnp.float32)
                   + b1_ref[...]
                   + jnp.dot(s4, ew1, preferred_element_type=jnp.float32))
    ea2_ref[...] = jnp.dot(s4, ew2, preferred_element_type=jnp.float32)
    normc_ref[...] = jnp.broadcast_to(norm, (BLK, 8))


def _dense2_body(z1_ref, p1a_ref, p1b_ref, normc_ref, w2_ref, b2_ref, ea2_ref,
                 ynorm2_ref, z2_ref):
    norm = normc_ref[:, 0:1]
    h1 = jax.nn.relu(z1_ref[...] + 2.0 * norm
                     * (p1a_ref[:, 0:64] + p1b_ref[:, 0:64]))
    w = w2_ref[...]
    y2 = jnp.dot(h1, w[2], preferred_element_type=jnp.float32)
    ynorm2_ref[...] = jnp.concatenate(
        [norm * y2, jnp.zeros((BLK, 64), jnp.float32)], axis=1)
    a = w[0] + w[1] - w[2]
    z2_ref[...] = (jnp.dot(h1, a, preferred_element_type=jnp.float32)
                   + b2_ref[...] + ea2_ref[...])


def _final_body(z2_ref, p2a_ref, p2b_ref, normc_ref, batch_ref, wl_ref, bl_ref,
                out_ref, acc_ref):
    i = pl.program_id(0)

    @pl.when(i == 0)
    def _init():
        acc_ref[...] = jnp.zeros((64, 128), jnp.float32)

    norm = normc_ref[:, 0:1]
    h2 = jax.nn.relu(z2_ref[...] + 2.0 * norm
                     * (p2a_ref[:, 0:64] + p2b_ref[:, 0:64]))
    h2c = jnp.concatenate([h2, jnp.ones((BLK, 64), jnp.float32)], axis=1)
    b = batch_ref[:, 0:1]
    g = lax.broadcasted_iota(jnp.int32, (1, 64), 1)
    m = (b == g).astype(jnp.float32)
    acc_ref[...] += lax.dot_general(
        m, h2c, (((0,), (0,)), ((), ())), preferred_element_type=jnp.float32)

    @pl.when(i == GRID - 1)
    def _fin():
        acc = acc_ref[...]
        cnt = jnp.maximum(acc[:, 64:65], 1.0)
        pooled = acc[:, 0:64] / cnt
        logits = (jnp.dot(pooled, wl_ref[...], preferred_element_type=jnp.float32)
                  + bl_ref[...])
        mx = jnp.max(logits, axis=1, keepdims=True)
        sh = logits - mx
        out_ref[...] = sh - jnp.log(jnp.sum(jnp.exp(sh), axis=1, keepdims=True))


def _row_spec(cols):
    return pl.BlockSpec((BLK, cols), lambda i: (i, 0))


def _full_spec(shape):
    nd = len(shape)
    return pl.BlockSpec(shape, lambda i: (0,) * nd)


_dense1 = pl.pallas_call(
    _dense1_body,
    grid=(GRID,),
    in_specs=[
        _row_spec(128), _row_spec(128), _row_spec(128),
        _full_spec((3, 128, 64)), _full_spec((1, 64)),
        _full_spec((3, 4, 64)), _full_spec((3, 4, 64)),
    ],
    out_specs=[_row_spec(128), _row_spec(64), _row_spec(64), _row_spec(8)],
    out_shape=[
        jax.ShapeDtypeStruct((N_PAD, 128), jnp.float32),
        jax.ShapeDtypeStruct((N_PAD, 64), jnp.float32),
        jax.ShapeDtypeStruct((N_PAD, 64), jnp.float32),
        jax.ShapeDtypeStruct((N_PAD, 8), jnp.float32),
    ],
)

_dense2 = pl.pallas_call(
    _dense2_body,
    grid=(GRID,),
    in_specs=[
        _row_spec(64), _row_spec(128), _row_spec(128), _row_spec(8),
        _full_spec((3, 64, 64)), _full_spec((1, 64)), _row_spec(64),
    ],
    out_specs=[_row_spec(128), _row_spec(64)],
    out_shape=[
        jax.ShapeDtypeStruct((N_PAD, 128), jnp.float32),
        jax.ShapeDtypeStruct((N_PAD, 64), jnp.float32),
    ],
)

_final = pl.pallas_call(
    _final_body,
    grid=(GRID,),
    in_specs=[
        _row_spec(64), _row_spec(128), _row_spec(128), _row_spec(8),
        pl.BlockSpec((BLK, 8), lambda i: (i, 0)),
        _full_spec((64, 4)), _full_spec((1, 4)),
    ],
    out_specs=pl.BlockSpec((64, 4), lambda i: (0, 0)),
    out_shape=jax.ShapeDtypeStruct((64, 4), jnp.float32),
    scratch_shapes=[pltpu.VMEM((64, 128), jnp.float32)],
)


@jax.jit
def kernel(x, edge_index, edge_attr, batch, W1, EW1, b1, W2, EW2, b2, Wl, bl):
    f32 = jnp.float32
    x_pad = jnp.pad(x, ((0, N_PAD - N_NODES), (0, 0)))
    pad_e = E_PAD - E_EDGES
    row = jnp.concatenate(
        [edge_index[0], jnp.full((pad_e,), N_NODES, jnp.int32)])
    col = jnp.concatenate(
        [edge_index[1], jnp.full((pad_e,), N_NODES, jnp.int32)])
    ridx = row.reshape(N_TILES, N_CHUNKS, CHUNK)
    cidx = col.reshape(N_TILES, N_CHUNKS, CHUNK)
    eb = jnp.concatenate(
        [edge_attr, jnp.ones((E_EDGES, 1), f32), jnp.zeros((E_EDGES, 123), f32)],
        axis=1)
    eb = jnp.pad(eb, ((0, pad_e), (0, 0))).reshape(
        N_TILES, N_CHUNKS, CHUNK, 128)
    batchc = jnp.broadcast_to(
        jnp.pad(batch, (0, N_PAD - N_NODES), constant_values=64)[:, None],
        (N_PAD, 8))
    b1r = b1.reshape(1, 64)
    b2r = b2.reshape(1, 64)
    blr = bl.reshape(1, 4)

    s5p = _attr_scatter()(eb, ridx)
    ynorm1, z1, ea2, normc = _dense1(x_pad, s5p[0], s5p[1], W1, b1r, EW1, EW2)
    p1 = _edge_scatter()(ynorm1, ridx, cidx)
    ynorm2, z2 = _dense2(z1, p1[0], p1[1], normc, W2, b2r, ea2)
    p2 = _edge_scatter()(ynorm2, ridx, cidx)
    return _final(z2, p2[0], p2[1], normc, batchc, Wl, blr)


# speedup vs baseline: 10.4839x; 1.6965x over previous
import functools
import jax
import jax.numpy as jnp
from jax import lax
from jax.experimental import pallas as pl
from jax.experimental.pallas import tpu as pltpu
from jax.experimental.pallas import tpu_sc as plsc

N_NODES = 10000
N_PAD = 10240
E_EDGES = 320000
E_PAD = 327680
N_TILES = 32
N_CHUNKS = 80
CHUNK = 128
ROWS_PER_TILE = N_PAD // 16
BLK = 1024
GRID = N_PAD // BLK

@functools.cache
def _sc_mesh():
    return plsc.VectorSubcoreMesh(core_axis_name="c", subcore_axis_name="s")


def _zero_vmem(buf, ncols):
    nj = ncols // 16
    def zrow(r, carry):
        for j in range(nj):
            buf[r, pl.ds(16 * j, 16)] = jnp.zeros((16,), jnp.float32)
        return carry
    lax.fori_loop(0, 128, zrow, 0)


def _edge_scatter_body(table_hbm, ridx_hbm, cidx_hbm, out_hbm,
                       acc_sh, rows_v, ridx_v, cidx_v, sem):
    ci = lax.axis_index("c")
    si = lax.axis_index("s")
    wid = ci * 16 + si

    _zero_vmem(rows_v, 128)
    def zchunk(k, carry):
        r = si * ROWS_PER_TILE + k * 128
        pltpu.sync_copy(rows_v, acc_sh.at[pl.ds(r, 128)])
        return carry
    lax.fori_loop(0, ROWS_PER_TILE // 128, zchunk, 0)
    plsc.subcore_barrier()

    pltpu.sync_copy(ridx_hbm.at[wid], ridx_v)
    pltpu.sync_copy(cidx_hbm.at[wid], cidx_v)

    def echunk(c, carry):
        pltpu.async_copy(table_hbm.at[ridx_v.at[c]], rows_v, sem).wait()
        pltpu.sync_copy(rows_v, acc_sh.at[cidx_v.at[c]], add=True)
        return carry
    lax.fori_loop(0, N_CHUNKS, echunk, 0)
    plsc.subcore_barrier()

    def xchunk(k, carry):
        r = si * ROWS_PER_TILE + k * 128
        pltpu.sync_copy(acc_sh.at[pl.ds(r, 128)], rows_v)
        pltpu.sync_copy(rows_v, out_hbm.at[ci, pl.ds(r, 128)])
        return carry
    lax.fori_loop(0, ROWS_PER_TILE // 128, xchunk, 0)


@functools.cache
def _edge_scatter():
    return pl.kernel(
        _edge_scatter_body,
        out_type=jax.ShapeDtypeStruct((2, N_PAD, 128), jnp.float32),
        mesh=_sc_mesh(),
        scratch_types=[
            pltpu.VMEM_SHARED((N_PAD, 128), jnp.float32),
            pltpu.VMEM((CHUNK, 128), jnp.float32),
            pltpu.VMEM((N_CHUNKS, CHUNK), jnp.int32),
            pltpu.VMEM((N_CHUNKS, CHUNK), jnp.int32),
            pltpu.SemaphoreType.DMA,
        ],
    )


def _attr_scatter_body(eb_hbm, ridx_hbm, out_hbm,
                       acc_sh, ebuf, ridx_v):
    ci = lax.axis_index("c")
    si = lax.axis_index("s")
    wid = ci * 16 + si

    _zero_vmem(ebuf, 128)
    def zchunk(k, carry):
        pltpu.sync_copy(ebuf, acc_sh.at[pl.ds(si * ROWS_PER_TILE + k * 128, 128)])
        return carry
    lax.fori_loop(0, ROWS_PER_TILE // 128, zchunk, 0)
    plsc.subcore_barrier()

    pltpu.sync_copy(ridx_hbm.at[wid], ridx_v)

    def echunk(c, carry):
        pltpu.sync_copy(eb_hbm.at[wid, c], ebuf)
        pltpu.sync_copy(ebuf, acc_sh.at[ridx_v.at[c]], add=True)
        return carry
    lax.fori_loop(0, N_CHUNKS, echunk, 0)
    plsc.subcore_barrier()

    def xchunk(k, carry):
        r = si * ROWS_PER_TILE + k * 128
        pltpu.sync_copy(acc_sh.at[pl.ds(r, 128)], ebuf)
        pltpu.sync_copy(ebuf, out_hbm.at[ci, pl.ds(r, 128)])
        return carry
    lax.fori_loop(0, ROWS_PER_TILE // 128, xchunk, 0)


@functools.cache
def _attr_scatter():
    return pl.kernel(
        _attr_scatter_body,
        out_type=jax.ShapeDtypeStruct((2, N_PAD, 128), jnp.float32),
        mesh=_sc_mesh(),
        scratch_types=[
            pltpu.VMEM_SHARED((N_PAD, 128), jnp.float32),
            pltpu.VMEM((CHUNK, 128), jnp.float32),
            pltpu.VMEM((N_CHUNKS, CHUNK), jnp.int32),
        ],
    )


def _dense1_body(x_ref, s5a_ref, s5b_ref, w1_ref, b1_ref, ew1_ref, ew2_ref,
                 ynorm_ref, z1_ref, ea2_ref, normc_ref):
    x = x_ref[...]
    s5 = s5a_ref[...] + s5b_ref[...]
    deg = s5[:, 4:5]
    norm = jnp.where(deg > 0, lax.rsqrt(jnp.maximum(deg, 1e-30)), 0.0)
    w = w1_ref[...]
    y = jnp.dot(x, w[2], preferred_element_type=jnp.float32)
    ynorm_ref[...] = jnp.concatenate(
        [norm * y, jnp.zeros((BLK, 64), jnp.float32)], axis=1)
    a = w[0] + w[1] - w[2]
    ew1 = ew1_ref[0] + ew1_ref[1] + ew1_ref[2]
    ew2 = ew2_ref[0] + ew2_ref[1] + ew2_ref[2]
    s4 = s5[:, 0:4]
    z1_ref[...] = (jnp.dot(x, a, preferred_element_type=jnp.float32)
                   + b1_ref[...]
                   + jnp.dot(s4, ew1, preferred_element_type=jnp.float32))
    ea2_ref[...] = jnp.dot(s4, ew2, preferred_element_type=jnp.float32)
    normc_ref[...] = jnp.broadcast_to(norm, (BLK, 8))


def _dense2_body(z1_ref, p1a_ref, p1b_ref, normc_ref, w2_ref, b2_ref, ea2_ref,
                 ynorm2_ref, z2_ref):
    norm = normc_ref[:, 0:1]
    h1 = jax.nn.relu(z1_ref[...] + 2.0 * norm
                     * (p1a_ref[:, 0:64] + p1b_ref[:, 0:64]))
    w = w2_ref[...]
    y2 = jnp.dot(h1, w[2], preferred_element_type=jnp.float32)
    ynorm2_ref[...] = jnp.concatenate(
        [norm * y2, jnp.zeros((BLK, 64), jnp.float32)], axis=1)
    a = w[0] + w[1] - w[2]
    z2_ref[...] = (jnp.dot(h1, a, preferred_element_type=jnp.float32)
                   + b2_ref[...] + ea2_ref[...])


def _final_body(z2_ref, p2a_ref, p2b_ref, normc_ref, batch_ref, wl_ref, bl_ref,
                out_ref, acc_ref):
    i = pl.program_id(0)

    @pl.when(i == 0)
    def _init():
        acc_ref[...] = jnp.zeros((64, 128), jnp.float32)

    norm = normc_ref[:, 0:1]
    h2 = jax.nn.relu(z2_ref[...] + 2.0 * norm
                     * (p2a_ref[:, 0:64] + p2b_ref[:, 0:64]))
    h2c = jnp.concatenate([h2, jnp.ones((BLK, 64), jnp.float32)], axis=1)
    b = batch_ref[:, 0:1]
    g = lax.broadcasted_iota(jnp.int32, (1, 64), 1)
    m = (b == g).astype(jnp.float32)
    acc_ref[...] += lax.dot_general(
        m, h2c, (((0,), (0,)), ((), ())), preferred_element_type=jnp.float32)

    @pl.when(i == GRID - 1)
    def _fin():
        acc = acc_ref[...]
        cnt = jnp.maximum(acc[:, 64:65], 1.0)
        pooled = acc[:, 0:64] / cnt
        logits = (jnp.dot(pooled, wl_ref[...], preferred_element_type=jnp.float32)
                  + bl_ref[...])
        mx = jnp.max(logits, axis=1, keepdims=True)
        sh = logits - mx
        out_ref[...] = sh - jnp.log(jnp.sum(jnp.exp(sh), axis=1, keepdims=True))


def _row_spec(cols):
    return pl.BlockSpec((BLK, cols), lambda i: (i, 0))


def _full_spec(shape):
    nd = len(shape)
    return pl.BlockSpec(shape, lambda i: (0,) * nd)


_dense1 = pl.pallas_call(
    _dense1_body,
    grid=(GRID,),
    in_specs=[
        _row_spec(128), _row_spec(128), _row_spec(128),
        _full_spec((3, 128, 64)), _full_spec((1, 64)),
        _full_spec((3, 4, 64)), _full_spec((3, 4, 64)),
    ],
    out_specs=[_row_spec(128), _row_spec(64), _row_spec(64), _row_spec(8)],
    out_shape=[
        jax.ShapeDtypeStruct((N_PAD, 128), jnp.float32),
        jax.ShapeDtypeStruct((N_PAD, 64), jnp.float32),
        jax.ShapeDtypeStruct((N_PAD, 64), jnp.float32),
        jax.ShapeDtypeStruct((N_PAD, 8), jnp.float32),
    ],
)

_dense2 = pl.pallas_call(
    _dense2_body,
    grid=(GRID,),
    in_specs=[
        _row_spec(64), _row_spec(128), _row_spec(128), _row_spec(8),
        _full_spec((3, 64, 64)), _full_spec((1, 64)), _row_spec(64),
    ],
    out_specs=[_row_spec(128), _row_spec(64)],
    out_shape=[
        jax.ShapeDtypeStruct((N_PAD, 128), jnp.float32),
        jax.ShapeDtypeStruct((N_PAD, 64), jnp.float32),
    ],
)

_final = pl.pallas_call(
    _final_body,
    grid=(GRID,),
    in_specs=[
        _row_spec(64), _row_spec(128), _row_spec(128), _row_spec(8),
        pl.BlockSpec((BLK, 8), lambda i: (i, 0)),
        _full_spec((64, 4)), _full_spec((1, 4)),
    ],
    out_specs=pl.BlockSpec((64, 4), lambda i: (0, 0)),
    out_shape=jax.ShapeDtypeStruct((64, 4), jnp.float32),
    scratch_shapes=[pltpu.VMEM((64, 128), jnp.float32)],
)


@jax.jit
def kernel(x, edge_index, edge_attr, batch, W1, EW1, b1, W2, EW2, b2, Wl, bl):
    f32 = jnp.float32
    x_pad = jnp.pad(x, ((0, N_PAD - N_NODES), (0, 0)))
    pad_e = E_PAD - E_EDGES
    dummy = N_NODES + (jnp.arange(pad_e, dtype=jnp.int32)
                       % (N_PAD - N_NODES))
    row = jnp.concatenate([edge_index[0], dummy])
    col = jnp.concatenate([edge_index[1], dummy])
    ridx = row.reshape(N_TILES, N_CHUNKS, CHUNK)
    cidx = col.reshape(N_TILES, N_CHUNKS, CHUNK)
    eb = jnp.concatenate(
        [edge_attr, jnp.ones((E_EDGES, 1), f32), jnp.zeros((E_EDGES, 123), f32)],
        axis=1)
    eb = jnp.pad(eb, ((0, pad_e), (0, 0))).reshape(
        N_TILES, N_CHUNKS, CHUNK, 128)
    batchc = jnp.broadcast_to(
        jnp.pad(batch, (0, N_PAD - N_NODES), constant_values=64)[:, None],
        (N_PAD, 8))
    b1r = b1.reshape(1, 64)
    b2r = b2.reshape(1, 64)
    blr = bl.reshape(1, 4)

    s5p = _attr_scatter()(eb, ridx)
    ynorm1, z1, ea2, normc = _dense1(x_pad, s5p[0], s5p[1], W1, b1r, EW1, EW2)
    p1 = _edge_scatter()(ynorm1, ridx, cidx)
    ynorm2, z2 = _dense2(z1, p1[0], p1[1], normc, W2, b2r, ea2)
    p2 = _edge_scatter()(ynorm2, ridx, cidx)
    return _final(z2, p2[0], p2[1], normc, batchc, Wl, blr)
